# baseline (speedup 1.0000x reference)
import functools

import jax
import jax.numpy as jnp
from jax import lax
from jax.experimental import pallas as pl
from jax.experimental.pallas import tpu as pltpu
from jax.experimental.pallas import tpu_sc as plsc

_NC = 2
_NS = 16
_NW = _NC * _NS

_H = 512
_W = 512
_E = 16
_N = _H * _W
_S = 1024
_PAD = 17
_TAB = _S * _PAD
_HPW = _H // _NW
_STRIP = _S // _NS

_CHH_A = 4
_NCH_A = _HPW // _CHH_A
_CHP_A = _CHH_A * _W

_CHH_C = 2
_NCH_C = _HPW // _CHH_C

_F5 = (_H, 2, 4, 8, 128)


def _sc_mesh():
    return plsc.VectorSubcoreMesh(core_axis_name="c", subcore_axis_name="s")


_SC_PARAMS = pltpu.CompilerParams(
    use_tc_tiling_on_sc=False, needs_layout_passes=False
)


@functools.partial(
    pl.kernel,
    out_type=(
        jax.ShapeDtypeStruct((_NC, _S, _E), jnp.float32),
        jax.ShapeDtypeStruct((_NC, _S), jnp.float32),
    ),
    mesh=_sc_mesh(),
    scratch_types=[
        pltpu.VMEM((_TAB,), jnp.float32),
        pltpu.VMEM((_S,), jnp.float32),
        pltpu.VMEM((2, _CHH_A, 2, 4, 8, 128), jnp.float32),
        pltpu.VMEM((2, 4, 8, 128), jnp.int32),
        pltpu.VMEM((_NS, _STRIP * _PAD), jnp.float32),
        pltpu.VMEM((_STRIP, _E), jnp.float32),
        pltpu.VMEM((_NS, _STRIP), jnp.float32),
        pltpu.VMEM((_STRIP,), jnp.float32),
        pltpu.VMEM_SHARED((_NS, _TAB), jnp.float32),
        pltpu.VMEM_SHARED((_NS, _S), jnp.float32),
        pltpu.SemaphoreType.DMA,
        pltpu.SemaphoreType.DMA,
        pltpu.SemaphoreType.DMA,
        pltpu.SemaphoreType.DMA,
    ],
    compiler_params=_SC_PARAMS,
)
def _segment_sums(fused_hbm, seg_hbm, sums_hbm, cnt_hbm, tab, cnt, fch, sch,
                  rbuf, obuf, cbuf, cacc, shared, sharedc,
                  semf0, semf1, sems0, sems1):
    cid = lax.axis_index("c")
    sid = lax.axis_index("s")
    wid = sid * _NC + cid
    h_base = wid * _HPW

    zero = jnp.zeros((16,), jnp.float32)
    ones = jnp.ones((16,), jnp.float32)

    @plsc.parallel_loop(0, _TAB // 16, unroll=4)
    def _(k):
        tab[pl.ds(k * 16, 16)] = zero

    @plsc.parallel_loop(0, _S // 16, unroll=4)
    def _(k):
        cnt[pl.ds(k * 16, 16)] = zero

    semf = (semf0, semf1)

    blk0 = h_base // 8
    cseg = pltpu.async_copy(seg_hbm.at[pl.ds(blk0, 2)], sch, sems0)

    def start(g):
        b = g % 2
        h0 = h_base + g * _CHH_A
        cf = pltpu.async_copy(fused_hbm.at[pl.ds(h0, _CHH_A)], fch.at[b], semf[b])
        return cf

    pending = start(0)
    cseg.wait()
    for g in range(_NCH_A):
        b = g % 2
        nxt = start(g + 1) if g + 1 < _NCH_A else None
        pending.wait()

        @plsc.parallel_loop(0, _CHP_A // 16, unroll=2)
        def _(q):
            lh = q >> 5
            w0 = (q & 31) * 16
            wt = w0 >> 7
            wl = w0 & 127
            r = g * _CHH_A + lh
            sv = sch[r >> 3, wt, r & 7, pl.ds(wl, 16)]
            svp = sv * _PAD
            for e in range(_E):
                vals = fch[b, lh, e >> 3, wt, e & 7, pl.ds(wl, 16)]
                plsc.addupdate_scatter(tab, [svp + e], vals)
            plsc.addupdate_scatter(cnt, [sv], ones)

        pending = nxt

    pltpu.sync_copy(tab, shared.at[sid])
    pltpu.sync_copy(cnt, sharedc.at[sid])
    plsc.subcore_barrier()

    strip = sid * _STRIP
    pltpu.sync_copy(shared.at[:, pl.ds(strip * _PAD, _STRIP * _PAD)], rbuf)
    pltpu.sync_copy(sharedc.at[:, pl.ds(strip, _STRIP)], cbuf)

    @plsc.parallel_loop(0, _STRIP, unroll=2)
    def _(r):
        acc = rbuf[0, pl.ds(r * _PAD, 16)]
        for t in range(1, _NS):
            acc = acc + rbuf[t, pl.ds(r * _PAD, 16)]
        obuf[r, :] = acc

    @plsc.parallel_loop(0, _STRIP // 16, unroll=2)
    def _(k):
        acc = cbuf[0, pl.ds(k * 16, 16)]
        for t in range(1, _NS):
            acc = acc + cbuf[t, pl.ds(k * 16, 16)]
        cacc[pl.ds(k * 16, 16)] = acc

    pltpu.sync_copy(obuf, sums_hbm.at[cid, pl.ds(strip, _STRIP)])
    pltpu.sync_copy(cacc, cnt_hbm.at[cid, pl.ds(strip, _STRIP)])


def _softmax_body(a_ref, w_ref):
    a = a_ref[...]
    m = jnp.max(a, axis=-1, keepdims=True)
    e = jnp.exp(a - m)
    w_ref[...] = e / jnp.sum(e, axis=-1, keepdims=True)


_softmax = pl.pallas_call(
    _softmax_body,
    out_shape=jax.ShapeDtypeStruct((_S, _S), jnp.float32),
)


def _attn_body(ps_ref, cnt_ref, w_ref, alpha_ref, t_ref, beta_ref):
    ps = ps_ref[...]
    sums = ps[0] + ps[1]
    cnt = cnt_ref[...]
    c = (cnt[0] + cnt[1])[:, None]
    means = sums / jnp.maximum(c, 1.0)
    w = w_ref[...]
    sm = jnp.dot(w, means, preferred_element_type=jnp.float32)
    al = jnp.clip(alpha_ref[0, 0], 0.0, 1.0)
    t_ref[...] = al * sm
    beta_ref[...] = jnp.full((1, _E), 1.0, jnp.float32) - al


_attn = pl.pallas_call(
    _attn_body,
    out_shape=[
        jax.ShapeDtypeStruct((_S, _E), jnp.float32),
        jax.ShapeDtypeStruct((1, _E), jnp.float32),
    ],
)


@functools.partial(
    pl.kernel,
    out_type=jax.ShapeDtypeStruct(_F5, jnp.float32),
    mesh=_sc_mesh(),
    scratch_types=[
        pltpu.VMEM((_S, _E), jnp.float32),
        pltpu.VMEM((_TAB,), jnp.float32),
        pltpu.VMEM((1, _E), jnp.float32),
        pltpu.VMEM((2, _CHH_C, 2, 4, 8, 128), jnp.float32),
        pltpu.VMEM((2, 4, 8, 128), jnp.int32),
        pltpu.VMEM((2, _CHH_C, 2, 4, 8, 128), jnp.float32),
        pltpu.SemaphoreType.DMA,
        pltpu.SemaphoreType.DMA,
        pltpu.SemaphoreType.DMA,
        pltpu.SemaphoreType.DMA,
        pltpu.SemaphoreType.DMA,
        pltpu.SemaphoreType.DMA,
    ],
    compiler_params=_SC_PARAMS,
)
def _gather_blend(fused_hbm, seg_hbm, t_hbm, beta_hbm, out_hbm,
                  tvc, tvp, bv, fch, sch, och,
                  semf0, semf1, sems0, sems1, semo0, semo1):
    cid = lax.axis_index("c")
    sid = lax.axis_index("s")
    wid = sid * _NC + cid
    h_base = wid * _HPW

    pltpu.sync_copy(t_hbm, tvc)
    pltpu.sync_copy(beta_hbm, bv)
    b = bv[0, :]

    @plsc.parallel_loop(0, _S, unroll=4)
    def _(s):
        tvp[pl.ds(s * _PAD, 16)] = tvc[s, :]

    semf = (semf0, semf1)
    semo = (semo0, semo1)

    blk0 = h_base // 8
    cseg = pltpu.async_copy(seg_hbm.at[pl.ds(blk0, 2)], sch, sems0)

    def start(g):
        k = g % 2
        cf = pltpu.async_copy(
            fused_hbm.at[pl.ds(h_base + g * _CHH_C, _CHH_C)], fch.at[k], semf[k]
        )
        return cf

    pending = start(0)
    cseg.wait()
    out_pending = [None, None]
    for g in range(_NCH_C):
        k = g % 2
        nxt = start(g + 1) if g + 1 < _NCH_C else None
        pending.wait()
        if out_pending[k] is not None:
            out_pending[k].wait()

        @plsc.parallel_loop(0, _CHH_C * _W // 16, unroll=2)
        def _(q):
            lh = q >> 5
            w0 = (q & 31) * 16
            wt = w0 >> 7
            wl = w0 & 127
            r = g * _CHH_C + lh
            sv = sch[r >> 3, wt, r & 7, pl.ds(wl, 16)]
            svp = sv * _PAD
            for e in range(_E):
                gathered = plsc.load_gather(tvp, [svp + e])
                och[k, lh, e >> 3, wt, e & 7, pl.ds(wl, 16)] = (
                    gathered + b * fch[k, lh, e >> 3, wt, e & 7, pl.ds(wl, 16)]
                )

        out_pending[k] = pltpu.async_copy(
            och.at[k], out_hbm.at[pl.ds(h_base + g * _CHH_C, _CHH_C)], semo[k]
        )
        pending = nxt

    for cp in out_pending:
        if cp is not None:
            cp.wait()


def kernel(fused_abundances, superpixel_segments, attention_matrix, alpha):
    H, W, E = fused_abundances.shape
    f5 = fused_abundances.reshape(_H, 4, 128, 2, 8).transpose(0, 3, 1, 4, 2)
    s5 = (
        superpixel_segments.astype(jnp.int32)
        .reshape(64, 8, 4, 128)
        .transpose(0, 2, 1, 3)
    )
    w = _softmax(attention_matrix)
    sums, cnts = _segment_sums(f5, s5)
    t_tab, beta = _attn(
        sums, cnts, w, jnp.reshape(alpha, (1, 1)).astype(jnp.float32)
    )
    o5 = _gather_blend(f5, s5, t_tab, beta)
    return o5.transpose(0, 2, 4, 1, 3).reshape(_H, _W, _E)


# speedup vs baseline: 1.0039x; 1.0039x over previous
import functools

import jax
import jax.numpy as jnp
from jax import lax
from jax.experimental import pallas as pl
from jax.experimental.pallas import tpu as pltpu
from jax.experimental.pallas import tpu_sc as plsc

_NC = 2
_NS = 16
_NW = _NC * _NS

_H = 512
_W = 512
_E = 16
_N = _H * _W
_S = 1024
_PAD = 17
_TAB = _S * _PAD
_HPW = _H // _NW
_STRIP = _S // _NS

_CHH_A = 4
_NCH_A = _HPW // _CHH_A
_CHP_A = _CHH_A * _W

_CHH_C = 2
_NCH_C = _HPW // _CHH_C

_F5 = (_H, 2, 4, 8, 128)


def _sc_mesh():
    return plsc.VectorSubcoreMesh(core_axis_name="c", subcore_axis_name="s")


_SC_PARAMS = pltpu.CompilerParams(
    use_tc_tiling_on_sc=False, needs_layout_passes=False
)


@functools.partial(
    pl.kernel,
    out_type=(
        jax.ShapeDtypeStruct((_NC, _S, _E), jnp.float32),
        jax.ShapeDtypeStruct((_NC, _S), jnp.float32),
    ),
    mesh=_sc_mesh(),
    scratch_types=[
        pltpu.VMEM((_TAB,), jnp.float32),
        pltpu.VMEM((_S,), jnp.float32),
        pltpu.VMEM((2, _CHH_A, 2, 4, 8, 128), jnp.float32),
        pltpu.VMEM((2, 4, 8, 128), jnp.int32),
        pltpu.VMEM((_NS, _STRIP * _PAD), jnp.float32),
        pltpu.VMEM((_STRIP, _E), jnp.float32),
        pltpu.VMEM((_NS, _STRIP), jnp.float32),
        pltpu.VMEM((_STRIP,), jnp.float32),
        pltpu.VMEM_SHARED((_NS, _TAB), jnp.float32),
        pltpu.VMEM_SHARED((_NS, _S), jnp.float32),
        pltpu.SemaphoreType.DMA,
        pltpu.SemaphoreType.DMA,
        pltpu.SemaphoreType.DMA,
        pltpu.SemaphoreType.DMA,
    ],
    compiler_params=_SC_PARAMS,
)
def _segment_sums(fused_hbm, seg_hbm, sums_hbm, cnt_hbm, tab, cnt, fch, sch,
                  rbuf, obuf, cbuf, cacc, shared, sharedc,
                  semf0, semf1, sems0, sems1):
    cid = lax.axis_index("c")
    sid = lax.axis_index("s")
    wid = sid * _NC + cid
    h_base = wid * _HPW

    zero = jnp.zeros((16,), jnp.float32)
    ones = jnp.ones((16,), jnp.float32)

    @plsc.parallel_loop(0, _TAB // 16, unroll=4)
    def _(k):
        tab[pl.ds(k * 16, 16)] = zero

    @plsc.parallel_loop(0, _S // 16, unroll=4)
    def _(k):
        cnt[pl.ds(k * 16, 16)] = zero

    semf = (semf0, semf1)

    blk0 = h_base // 8
    cseg = pltpu.async_copy(seg_hbm.at[pl.ds(blk0, 2)], sch, sems0)

    def start(g):
        b = g % 2
        h0 = h_base + g * _CHH_A
        cf = pltpu.async_copy(fused_hbm.at[pl.ds(h0, _CHH_A)], fch.at[b], semf[b])
        return cf

    pending = start(0)
    cseg.wait()
    for g in range(_NCH_A):
        b = g % 2
        nxt = start(g + 1) if g + 1 < _NCH_A else None
        pending.wait()

        @plsc.parallel_loop(0, _CHP_A // 16, unroll=2)
        def _(q):
            lh = q >> 5
            w0 = (q & 31) * 16
            wt = w0 >> 7
            wl = w0 & 127
            r = g * _CHH_A + lh
            sv = sch[r >> 3, wt, r & 7, pl.ds(wl, 16)]
            svp = sv * _PAD
            for e in range(_E):
                vals = fch[b, lh, e >> 3, wt, e & 7, pl.ds(wl, 16)]
                plsc.addupdate_scatter(tab, [svp + e], vals)
            plsc.addupdate_scatter(cnt, [sv], ones)

        pending = nxt

    pltpu.sync_copy(tab, shared.at[sid])
    pltpu.sync_copy(cnt, sharedc.at[sid])
    plsc.subcore_barrier()

    strip = sid * _STRIP
    pltpu.sync_copy(shared.at[:, pl.ds(strip * _PAD, _STRIP * _PAD)], rbuf)
    pltpu.sync_copy(sharedc.at[:, pl.ds(strip, _STRIP)], cbuf)

    @plsc.parallel_loop(0, _STRIP, unroll=2)
    def _(r):
        acc = rbuf[0, pl.ds(r * _PAD, 16)]
        for t in range(1, _NS):
            acc = acc + rbuf[t, pl.ds(r * _PAD, 16)]
        obuf[r, :] = acc

    @plsc.parallel_loop(0, _STRIP // 16, unroll=2)
    def _(k):
        acc = cbuf[0, pl.ds(k * 16, 16)]
        for t in range(1, _NS):
            acc = acc + cbuf[t, pl.ds(k * 16, 16)]
        cacc[pl.ds(k * 16, 16)] = acc

    pltpu.sync_copy(obuf, sums_hbm.at[cid, pl.ds(strip, _STRIP)])
    pltpu.sync_copy(cacc, cnt_hbm.at[cid, pl.ds(strip, _STRIP)])


def _attn_body(ps_ref, cnt_ref, a_ref, alpha_ref, t_ref, beta_ref):
    ps = ps_ref[...]
    sums = ps[0] + ps[1]
    cnt = cnt_ref[...]
    c = (cnt[0] + cnt[1])[:, None]
    means = sums / jnp.maximum(c, 1.0)
    a = a_ref[...]
    m = jnp.max(a, axis=-1, keepdims=True)
    e = jnp.exp(a - m)
    w = e / jnp.sum(e, axis=-1, keepdims=True)
    sm = jnp.dot(w, means, preferred_element_type=jnp.float32)
    al = jnp.clip(alpha_ref[0, 0], 0.0, 1.0)
    t_ref[...] = al * sm
    beta_ref[...] = jnp.full((1, _E), 1.0, jnp.float32) - al


_attn = pl.pallas_call(
    _attn_body,
    out_shape=[
        jax.ShapeDtypeStruct((_S, _E), jnp.float32),
        jax.ShapeDtypeStruct((1, _E), jnp.float32),
    ],
)


@functools.partial(
    pl.kernel,
    out_type=jax.ShapeDtypeStruct(_F5, jnp.float32),
    mesh=_sc_mesh(),
    scratch_types=[
        pltpu.VMEM((_S, _E), jnp.float32),
        pltpu.VMEM((_TAB,), jnp.float32),
        pltpu.VMEM((1, _E), jnp.float32),
        pltpu.VMEM((2, _CHH_C, 2, 4, 8, 128), jnp.float32),
        pltpu.VMEM((2, 4, 8, 128), jnp.int32),
        pltpu.VMEM((2, _CHH_C, 2, 4, 8, 128), jnp.float32),
        pltpu.SemaphoreType.DMA,
        pltpu.SemaphoreType.DMA,
        pltpu.SemaphoreType.DMA,
        pltpu.SemaphoreType.DMA,
        pltpu.SemaphoreType.DMA,
        pltpu.SemaphoreType.DMA,
    ],
    compiler_params=_SC_PARAMS,
)
def _gather_blend(fused_hbm, seg_hbm, t_hbm, beta_hbm, out_hbm,
                  tvc, tvp, bv, fch, sch, och,
                  semf0, semf1, sems0, sems1, semo0, semo1):
    cid = lax.axis_index("c")
    sid = lax.axis_index("s")
    wid = sid * _NC + cid
    h_base = wid * _HPW

    pltpu.sync_copy(t_hbm, tvc)
    pltpu.sync_copy(beta_hbm, bv)
    b = bv[0, :]

    @plsc.parallel_loop(0, _S, unroll=4)
    def _(s):
        tvp[pl.ds(s * _PAD, 16)] = tvc[s, :]

    semf = (semf0, semf1)
    semo = (semo0, semo1)

    blk0 = h_base // 8
    cseg = pltpu.async_copy(seg_hbm.at[pl.ds(blk0, 2)], sch, sems0)

    def start(g):
        k = g % 2
        cf = pltpu.async_copy(
            fused_hbm.at[pl.ds(h_base + g * _CHH_C, _CHH_C)], fch.at[k], semf[k]
        )
        return cf

    pending = start(0)
    cseg.wait()
    out_pending = [None, None]
    for g in range(_NCH_C):
        k = g % 2
        nxt = start(g + 1) if g + 1 < _NCH_C else None
        pending.wait()
        if out_pending[k] is not None:
            out_pending[k].wait()

        @plsc.parallel_loop(0, _CHH_C * _W // 16, unroll=2)
        def _(q):
            lh = q >> 5
            w0 = (q & 31) * 16
            wt = w0 >> 7
            wl = w0 & 127
            r = g * _CHH_C + lh
            sv = sch[r >> 3, wt, r & 7, pl.ds(wl, 16)]
            svp = sv * _PAD
            for e in range(_E):
                gathered = plsc.load_gather(tvp, [svp + e])
                och[k, lh, e >> 3, wt, e & 7, pl.ds(wl, 16)] = (
                    gathered + b * fch[k, lh, e >> 3, wt, e & 7, pl.ds(wl, 16)]
                )

        out_pending[k] = pltpu.async_copy(
            och.at[k], out_hbm.at[pl.ds(h_base + g * _CHH_C, _CHH_C)], semo[k]
        )
        pending = nxt

    for cp in out_pending:
        if cp is not None:
            cp.wait()


def kernel(fused_abundances, superpixel_segments, attention_matrix, alpha):
    H, W, E = fused_abundances.shape
    f5 = fused_abundances.reshape(_H, 4, 128, 2, 8).transpose(0, 3, 1, 4, 2)
    s5 = (
        superpixel_segments.astype(jnp.int32)
        .reshape(64, 8, 4, 128)
        .transpose(0, 2, 1, 3)
    )
    sums, cnts = _segment_sums(f5, s5)
    t_tab, beta = _attn(
        sums, cnts, attention_matrix, jnp.reshape(alpha, (1, 1)).astype(jnp.float32)
    )
    o5 = _gather_blend(f5, s5, t_tab, beta)
    return o5.transpose(0, 2, 4, 1, 3).reshape(_H, _W, _E)
